# baseline (speedup 1.0000x reference)
import functools

import numpy as np

import jax
import jax.numpy as jnp
from jax import lax
from jax.experimental import pallas as pl
from jax.experimental.pallas import tpu as pltpu
from jax.experimental.pallas import tpu_sc as plsc

_D = 256
_K = 16
_NC = 2
_NS = 16
_NW = _NC * _NS
_BP = 10240
_BW = _BP // _NW
_CH = 8
_NCHUNK = _BW // _CH
_NG = _D // 16


def _sc_body(nodes_hbm, nidx_hbm, emb_hbm, embb_hbm, self_hbm, neigh_hbm,
             nodes_v, nids2_v, nidsf_v, rows0_v, rows1_v, obuf0_v, obuf1_v,
             selfbuf_v, sem_n, sem_s, sem_g0, sem_g1, sem_o0, sem_o1):
    c = lax.axis_index("c")
    s = lax.axis_index("s")
    wid = c * _NS + s
    base = wid * _BW

    with jax.named_scope("sc_nids"):
        pltpu.sync_copy(nodes_hbm.at[pl.ds(base, _BW)], nodes_v)
        for k in range(4):
            pltpu.async_copy(
                nidx_hbm.at[nodes_v.at[pl.ds(k * 80, 80)]], nids2_v, sem_n
            ).wait()

            def _flat(i, _):
                nidsf_v[pl.ds((k * 80 + i) * _K, _K)] = (
                    nids2_v[i, pl.ds(0, _K)])
                return 0
            lax.fori_loop(0, 80, _flat, 0)

    with jax.named_scope("sc_self"):
        for k in range(4):
            pltpu.async_copy(
                emb_hbm.at[nodes_v.at[pl.ds(k * 80, 80)]], selfbuf_v, sem_s
            ).wait()
            pltpu.sync_copy(selfbuf_v, self_hbm.at[pl.ds(base + k * 80, 80)])

    rows = (rows0_v, rows1_v)
    obufs = (obuf0_v, obuf1_v)
    sems = (sem_g0, sem_g1)
    sems_o = (sem_o0, sem_o1)

    def _fire(ci, b):
        return pltpu.async_copy(
            embb_hbm.at[nidsf_v.at[pl.ds(ci * (_CH * _K), _CH * _K)]],
            rows[b], sems[b])

    with jax.named_scope("sc_prime"):
        _fire(0, 0)
        _fire(1, 1)

    def _pair(p, _):
        for b in range(2):
            ci = p * 2 + b
            @pl.when(ci >= 1)
            def _():
                pltpu.async_copy(
                    obufs[1 - b],
                    neigh_hbm.at[pl.ds(base + (ci - 1) * _CH, _CH)],
                    sems_o[1 - b])
            pltpu.make_async_copy(
                embb_hbm.at[nidsf_v.at[pl.ds(ci * (_CH * _K), _CH * _K)]],
                rows[b], sems[b]).wait()
            @pl.when(ci >= 2)
            def _():
                pltpu.make_async_copy(
                    obufs[b], neigh_hbm.at[pl.ds(base, _CH)], sems_o[b]
                ).wait()

            def _node(j, _):
                r0 = j * _K
                for cc in range(_D // 32):

                    def _tree(v):
                        while len(v) > 1:
                            v = [v[2 * i] + v[2 * i + 1]
                                 for i in range(len(v) // 2)]
                        return v[0]
                    evens, odds = [], []
                    for r in range(_K):
                        w = rows[b][r0 + r, pl.ds(cc * 16, 16)]
                        e, o = plsc.unpack(
                            plsc.bitcast(w, jnp.bfloat16),
                            format=plsc.PackFormat.INTERLEAVED)
                        evens.append(e)
                        odds.append(o)
                    obufs[b][j, pl.ds(cc * 32, 16)] = _tree(evens)
                    obufs[b][j, pl.ds(cc * 32 + 16, 16)] = _tree(odds)
                return 0
            lax.fori_loop(0, _CH, _node, 0)

            @pl.when(ci + 2 < _NCHUNK)
            def _():
                _fire(ci + 2, b)
        return 0
    with jax.named_scope("sc_main"):
        lax.fori_loop(0, _NCHUNK // 2, _pair, 0)

    with jax.named_scope("sc_drain"):
        pltpu.make_async_copy(
            obufs[0], neigh_hbm.at[pl.ds(base, _CH)], sems_o[0]).wait()
        plsc.subcore_barrier()
        pltpu.sync_copy(
            obufs[1],
            neigh_hbm.at[pl.ds(base + (_NCHUNK - 1) * _CH, _CH)])


def _sc_gather(nodes_p, nidx_p, emb, emb_bf):
    mesh = plsc.VectorSubcoreMesh(core_axis_name="c", subcore_axis_name="s")
    f = pl.kernel(
        _sc_body,
        out_type=(
            jax.ShapeDtypeStruct((_BP, _D), jnp.float32),
            jax.ShapeDtypeStruct((_BP, _D), jnp.float32),
        ),
        mesh=mesh,
        compiler_params=pltpu.CompilerParams(needs_layout_passes=False),
        scratch_types=[
            pltpu.VMEM((_BW,), jnp.int32),
            pltpu.VMEM((80, 128), jnp.int32),
            pltpu.VMEM((_BW * _K,), jnp.int32),
            pltpu.VMEM((_CH * _K, _D // 2), jnp.int32),
            pltpu.VMEM((_CH * _K, _D // 2), jnp.int32),
            pltpu.VMEM((_CH, _D), jnp.float32),
            pltpu.VMEM((_CH, _D), jnp.float32),
            pltpu.VMEM((80, _D), jnp.float32),
            pltpu.SemaphoreType.DMA,
            pltpu.SemaphoreType.DMA,
            pltpu.SemaphoreType.DMA,
            pltpu.SemaphoreType.DMA,
            pltpu.SemaphoreType.DMA,
            pltpu.SemaphoreType.DMA,
        ],
    )
    return f(nodes_p, nidx_p, emb, emb_bf)


def _tc_body(w1_ref, w2_ref, xs_ref, xn_ref, o_ref):
    a = lax.dot_general(w1_ref[...], xs_ref[...],
                        (((1,), (1,)), ((), ())),
                        preferred_element_type=jnp.float32)
    b = lax.dot_general(w2_ref[...], xn_ref[...],
                        (((1,), (1,)), ((), ())),
                        preferred_element_type=jnp.float32)
    o_ref[...] = jnp.maximum(a + b, 0.0)


def _tc_combine(w1, w2, xs, xn, n_out):
    blk = 2048
    grid = _BP // blk
    return pl.pallas_call(
        _tc_body,
        grid=(grid,),
        in_specs=[
            pl.BlockSpec((_D, _D), lambda i: (0, 0)),
            pl.BlockSpec((_D, _D), lambda i: (0, 0)),
            pl.BlockSpec((blk, _D), lambda i: (i, 0)),
            pl.BlockSpec((blk, _D), lambda i: (i, 0)),
        ],
        out_specs=pl.BlockSpec((_D, blk), lambda i: (0, i)),
        out_shape=jax.ShapeDtypeStruct((_D, n_out), jnp.float32),
    )(w1, w2, xs, xn)


_PERM = np.concatenate(
    [c * 32 + np.concatenate([np.arange(16) * 2, np.arange(16) * 2 + 1])
     for c in range(_D // 32)])


def kernel(nodes, emb, neigh_idx, W):
    B = nodes.shape[0]
    nodes32 = nodes.astype(jnp.int32)
    nidx32 = neigh_idx.astype(jnp.int32)
    nidx_p = jnp.pad(nidx32, ((0, 0), (0, 128 - _K)))
    nodes_p = jnp.zeros((_BP,), jnp.int32).at[:B].set(nodes32)
    emb_bf = lax.bitcast_convert_type(
        emb.astype(jnp.bfloat16).reshape(emb.shape[0], _D // 2, 2),
        jnp.int32)
    self_f, neigh_s = _sc_gather(nodes_p, nidx_p, emb, emb_bf)
    w1 = W[:, :_D]
    w2 = (W[:, _D:] * (1.0 / _K))[:, _PERM]
    return _tc_combine(w1, w2, self_f, neigh_s, B)


# speedup vs baseline: 1.2519x; 1.2519x over previous
import functools

import numpy as np

import jax
import jax.numpy as jnp
from jax import lax
from jax.experimental import pallas as pl
from jax.experimental.pallas import tpu as pltpu
from jax.experimental.pallas import tpu_sc as plsc

_D = 256
_K = 16
_NC = 2
_NS = 16
_NW = _NC * _NS
_BP = 10240
_BW = _BP // _NW
_CH = 8
_NCHUNK = _BW // _CH
_NG = _D // 16


def _sc_body(nodes_hbm, nidx_hbm, emb_hbm, self_hbm, neigh_hbm, embb_hbm,
             nodes_v, nids2_v, nidsf_v, rows0_v, rows1_v, obuf0_v, obuf1_v,
             selfbuf_v, pbin_v, pbout_v,
             sem_n, sem_s, sem_g0, sem_g1, sem_o0, sem_o1):
    c = lax.axis_index("c")
    s = lax.axis_index("s")
    wid = c * _NS + s
    base = wid * _BW

    n_chunks = emb_hbm.shape[0] // 40
    with jax.named_scope("sc_pack"):
        def _pk(j, _):
            cid = j * _NS + s

            @pl.when(cid < n_chunks)
            def _():
                r0 = cid * 40
                pltpu.sync_copy(emb_hbm.at[pl.ds(r0, 40)], pbin_v)

                def _prow(i, _):
                    for cc in range(_D // 32):
                        a = pbin_v[i, pl.ds(cc * 32, 16)]
                        bq = pbin_v[i, pl.ds(cc * 32 + 16, 16)]
                        pk = plsc.pack(a, bq,
                                       format=plsc.PackFormat.INTERLEAVED)
                        pbout_v[i, pl.ds(cc * 16, 16)] = plsc.bitcast(
                            pk, jnp.int32)
                    return 0
                lax.fori_loop(0, 40, _prow, 0)
                pltpu.sync_copy(pbout_v, embb_hbm.at[c, pl.ds(r0, 40)])
            return 0
        lax.fori_loop(0, (n_chunks + _NS - 1) // _NS, _pk, 0)
        plsc.subcore_barrier()

    with jax.named_scope("sc_nids"):
        pltpu.sync_copy(nodes_hbm.at[pl.ds(base, _BW)], nodes_v)
        for k in range(4):
            pltpu.async_copy(
                nidx_hbm.at[nodes_v.at[pl.ds(k * 80, 80)]], nids2_v, sem_n
            ).wait()

            def _flat(i, _):
                nidsf_v[pl.ds((k * 80 + i) * _K, _K)] = (
                    nids2_v[i, pl.ds(0, _K)])
                return 0
            lax.fori_loop(0, 80, _flat, 0)

    with jax.named_scope("sc_self"):
        for k in range(4):
            pltpu.async_copy(
                emb_hbm.at[nodes_v.at[pl.ds(k * 80, 80)]], selfbuf_v, sem_s
            ).wait()
            pltpu.sync_copy(selfbuf_v, self_hbm.at[pl.ds(base + k * 80, 80)])

    rows = (rows0_v, rows1_v)
    obufs = (obuf0_v, obuf1_v)
    sems = (sem_g0, sem_g1)
    sems_o = (sem_o0, sem_o1)

    def _fire(ci, b):
        return pltpu.async_copy(
            embb_hbm.at[c].at[nidsf_v.at[pl.ds(ci * (_CH * _K), _CH * _K)]],
            rows[b], sems[b])

    with jax.named_scope("sc_prime"):
        _fire(0, 0)
        _fire(1, 1)

    def _pair(p, _):
        for b in range(2):
            ci = p * 2 + b
            @pl.when(ci >= 1)
            def _():
                pltpu.async_copy(
                    obufs[1 - b],
                    neigh_hbm.at[pl.ds(base + (ci - 1) * _CH, _CH)],
                    sems_o[1 - b])
            pltpu.make_async_copy(
                embb_hbm.at[c].at[
                    nidsf_v.at[pl.ds(ci * (_CH * _K), _CH * _K)]],
                rows[b], sems[b]).wait()
            @pl.when(ci >= 2)
            def _():
                pltpu.make_async_copy(
                    obufs[b], neigh_hbm.at[pl.ds(base, _CH)], sems_o[b]
                ).wait()

            def _node(j, _):
                r0 = j * _K
                for cc in range(_D // 32):

                    def _tree(v):
                        while len(v) > 1:
                            v = [v[2 * i] + v[2 * i + 1]
                                 for i in range(len(v) // 2)]
                        return v[0]
                    evens, odds = [], []
                    for r in range(_K):
                        w = rows[b][r0 + r, pl.ds(cc * 16, 16)]
                        e, o = plsc.unpack(
                            plsc.bitcast(w, jnp.bfloat16),
                            format=plsc.PackFormat.INTERLEAVED)
                        evens.append(e)
                        odds.append(o)
                    obufs[b][j, pl.ds(cc * 32, 16)] = _tree(evens)
                    obufs[b][j, pl.ds(cc * 32 + 16, 16)] = _tree(odds)
                return 0
            lax.fori_loop(0, _CH, _node, 0)

            @pl.when(ci + 2 < _NCHUNK)
            def _():
                _fire(ci + 2, b)
        return 0
    with jax.named_scope("sc_main"):
        lax.fori_loop(0, _NCHUNK // 2, _pair, 0)

    with jax.named_scope("sc_drain"):
        pltpu.make_async_copy(
            obufs[0], neigh_hbm.at[pl.ds(base, _CH)], sems_o[0]).wait()
        pl.delay(256)
        pltpu.sync_copy(
            obufs[1],
            neigh_hbm.at[pl.ds(base + (_NCHUNK - 1) * _CH, _CH)])


def _sc_gather(nodes_p, nidx_p, emb):
    mesh = plsc.VectorSubcoreMesh(core_axis_name="c", subcore_axis_name="s")
    f = pl.kernel(
        _sc_body,
        out_type=(
            jax.ShapeDtypeStruct((_BP, _D), jnp.float32),
            jax.ShapeDtypeStruct((_BP, _D), jnp.float32),
            jax.ShapeDtypeStruct((_NC, emb.shape[0], _D // 2), jnp.int32),
        ),
        mesh=mesh,
        compiler_params=pltpu.CompilerParams(needs_layout_passes=False),
        scratch_types=[
            pltpu.VMEM((_BW,), jnp.int32),
            pltpu.VMEM((80, 128), jnp.int32),
            pltpu.VMEM((_BW * _K,), jnp.int32),
            pltpu.VMEM((_CH * _K, _D // 2), jnp.int32),
            pltpu.VMEM((_CH * _K, _D // 2), jnp.int32),
            pltpu.VMEM((_CH, _D), jnp.float32),
            pltpu.VMEM((_CH, _D), jnp.float32),
            pltpu.VMEM((80, _D), jnp.float32),
            pltpu.VMEM((40, _D), jnp.float32),
            pltpu.VMEM((40, _D // 2), jnp.int32),
            pltpu.SemaphoreType.DMA,
            pltpu.SemaphoreType.DMA,
            pltpu.SemaphoreType.DMA,
            pltpu.SemaphoreType.DMA,
            pltpu.SemaphoreType.DMA,
            pltpu.SemaphoreType.DMA,
        ],
    )
    return f(nodes_p, nidx_p, emb)[:2]


def _tc_body(w1_ref, w2_ref, xs_ref, xn_ref, o_ref):
    a = lax.dot_general(w1_ref[...], xs_ref[...],
                        (((1,), (1,)), ((), ())),
                        preferred_element_type=jnp.float32)
    b = lax.dot_general(w2_ref[...], xn_ref[...],
                        (((1,), (1,)), ((), ())),
                        preferred_element_type=jnp.float32)
    o_ref[...] = jnp.maximum(a + b, 0.0)


def _tc_combine(w1, w2, xs, xn, n_out):
    blk = 2048
    grid = _BP // blk
    return pl.pallas_call(
        _tc_body,
        grid=(grid,),
        in_specs=[
            pl.BlockSpec((_D, _D), lambda i: (0, 0)),
            pl.BlockSpec((_D, _D), lambda i: (0, 0)),
            pl.BlockSpec((blk, _D), lambda i: (i, 0)),
            pl.BlockSpec((blk, _D), lambda i: (i, 0)),
        ],
        out_specs=pl.BlockSpec((_D, blk), lambda i: (0, i)),
        out_shape=jax.ShapeDtypeStruct((_D, n_out), jnp.float32),
    )(w1, w2, xs, xn)


def kernel(nodes, emb, neigh_idx, W):
    B = nodes.shape[0]
    nodes32 = nodes.astype(jnp.int32)
    nidx32 = neigh_idx.astype(jnp.int32)
    nidx_p = jnp.pad(nidx32, ((0, 0), (0, 128 - _K)))
    nodes_p = jnp.zeros((_BP,), jnp.int32).at[:B].set(nodes32)
    self_f, neigh_s = _sc_gather(nodes_p, nidx_p, emb)
    w1 = W[:, :_D]
    w2 = W[:, _D:] * (1.0 / _K)
    return _tc_combine(w1, w2, self_f, neigh_s, B)


# speedup vs baseline: 1.3108x; 1.0471x over previous
import functools

import numpy as np

import jax
import jax.numpy as jnp
from jax import lax
from jax.experimental import pallas as pl
from jax.experimental.pallas import tpu as pltpu
from jax.experimental.pallas import tpu_sc as plsc

_D = 256
_K = 16
_NC = 2
_NS = 16
_NW = _NC * _NS
_BP = 10240
_BW = _BP // _NW
_CH = 8
_NCHUNK = _BW // _CH
_NG = _D // 16


def _sc_body(nodes_hbm, nidx_hbm, emb_hbm, self_hbm, neigh_hbm, embb_hbm,
             nodes_v, nids2_v, nidsf_v, rows0_v, rows1_v, obuf0_v, obuf1_v,
             selfbuf_v, pbin_v, pbout_v,
             sem_n, sem_s, sem_g0, sem_g1, sem_o0, sem_o1):
    c = lax.axis_index("c")
    s = lax.axis_index("s")
    wid = c * _NS + s
    base = wid * _BW

    n_chunks = emb_hbm.shape[0] // 40
    with jax.named_scope("sc_pack"):
        def _pk(j, _):
            cid = j * _NS + s

            @pl.when(cid < n_chunks)
            def _():
                r0 = cid * 40
                pltpu.sync_copy(emb_hbm.at[pl.ds(r0, 40)], pbin_v)

                def _prow(i, _):
                    for cc in range(_D // 32):
                        a = pbin_v[i, pl.ds(cc * 32, 16)]
                        bq = pbin_v[i, pl.ds(cc * 32 + 16, 16)]
                        pk = plsc.pack(a, bq,
                                       format=plsc.PackFormat.INTERLEAVED)
                        pbout_v[i, pl.ds(cc * 16, 16)] = plsc.bitcast(
                            pk, jnp.int32)
                    return 0
                lax.fori_loop(0, 40, _prow, 0)
                pltpu.sync_copy(pbout_v, embb_hbm.at[c, pl.ds(r0, 40)])
            return 0
        lax.fori_loop(0, (n_chunks + _NS - 1) // _NS, _pk, 0)

    with jax.named_scope("sc_nids"):
        pltpu.sync_copy(nodes_hbm.at[pl.ds(base, _BW)], nodes_v)
        for k in range(4):
            pltpu.async_copy(
                nidx_hbm.at[nodes_v.at[pl.ds(k * 80, 80)]], nids2_v, sem_n
            ).wait()

            def _flat(i, _):
                nidsf_v[pl.ds((k * 80 + i) * _K, _K)] = (
                    nids2_v[i, pl.ds(0, _K)])
                return 0
            lax.fori_loop(0, 80, _flat, 0)

    with jax.named_scope("sc_self"):
        for k in range(4):
            pltpu.async_copy(
                emb_hbm.at[nodes_v.at[pl.ds(k * 80, 80)]], selfbuf_v, sem_s
            ).wait()
            pltpu.sync_copy(selfbuf_v, self_hbm.at[pl.ds(base + k * 80, 80)])

    rows = (rows0_v, rows1_v)
    obufs = (obuf0_v, obuf1_v)
    sems = (sem_g0, sem_g1)
    sems_o = (sem_o0, sem_o1)

    def _fire(ci, b):
        return pltpu.async_copy(
            embb_hbm.at[c].at[nidsf_v.at[pl.ds(ci * (_CH * _K), _CH * _K)]],
            rows[b], sems[b])

    with jax.named_scope("sc_prime"):
        plsc.subcore_barrier()
        _fire(0, 0)
        _fire(1, 1)

    def _pair(p, _):
        for b in range(2):
            ci = p * 2 + b
            @pl.when(ci >= 1)
            def _():
                pltpu.async_copy(
                    obufs[1 - b],
                    neigh_hbm.at[pl.ds(base + (ci - 1) * _CH, _CH)],
                    sems_o[1 - b])
            pltpu.make_async_copy(
                embb_hbm.at[c].at[
                    nidsf_v.at[pl.ds(ci * (_CH * _K), _CH * _K)]],
                rows[b], sems[b]).wait()
            @pl.when(ci >= 2)
            def _():
                pltpu.make_async_copy(
                    obufs[b], neigh_hbm.at[pl.ds(base, _CH)], sems_o[b]
                ).wait()

            def _node(j, _):
                r0 = j * _K
                for cc in range(_D // 32):

                    def _tree(v):
                        while len(v) > 1:
                            v = [v[2 * i] + v[2 * i + 1]
                                 for i in range(len(v) // 2)]
                        return v[0]
                    evens, odds = [], []
                    for r in range(_K):
                        w = rows[b][r0 + r, pl.ds(cc * 16, 16)]
                        e, o = plsc.unpack(
                            plsc.bitcast(w, jnp.bfloat16),
                            format=plsc.PackFormat.INTERLEAVED)
                        evens.append(e)
                        odds.append(o)
                    obufs[b][j, pl.ds(cc * 32, 16)] = _tree(evens)
                    obufs[b][j, pl.ds(cc * 32 + 16, 16)] = _tree(odds)
                return 0
            lax.fori_loop(0, _CH, _node, 0)

            @pl.when(ci + 2 < _NCHUNK)
            def _():
                _fire(ci + 2, b)
        return 0
    with jax.named_scope("sc_main"):
        lax.fori_loop(0, _NCHUNK // 2, _pair, 0)

    with jax.named_scope("sc_drain"):
        pltpu.make_async_copy(
            obufs[0], neigh_hbm.at[pl.ds(base, _CH)], sems_o[0]).wait()
        pl.delay(256)
        pltpu.sync_copy(
            obufs[1],
            neigh_hbm.at[pl.ds(base + (_NCHUNK - 1) * _CH, _CH)])


def _sc_gather(nodes_p, nidx_p, emb):
    mesh = plsc.VectorSubcoreMesh(core_axis_name="c", subcore_axis_name="s")
    f = pl.kernel(
        _sc_body,
        out_type=(
            jax.ShapeDtypeStruct((_BP, _D), jnp.float32),
            jax.ShapeDtypeStruct((_BP, _D), jnp.float32),
            jax.ShapeDtypeStruct((_NC, emb.shape[0], _D // 2), jnp.int32),
        ),
        mesh=mesh,
        compiler_params=pltpu.CompilerParams(needs_layout_passes=False),
        scratch_types=[
            pltpu.VMEM((_BW,), jnp.int32),
            pltpu.VMEM((80, 128), jnp.int32),
            pltpu.VMEM((_BW * _K,), jnp.int32),
            pltpu.VMEM((_CH * _K, _D // 2), jnp.int32),
            pltpu.VMEM((_CH * _K, _D // 2), jnp.int32),
            pltpu.VMEM((_CH, _D), jnp.float32),
            pltpu.VMEM((_CH, _D), jnp.float32),
            pltpu.VMEM((80, _D), jnp.float32),
            pltpu.VMEM((40, _D), jnp.float32),
            pltpu.VMEM((40, _D // 2), jnp.int32),
            pltpu.SemaphoreType.DMA,
            pltpu.SemaphoreType.DMA,
            pltpu.SemaphoreType.DMA,
            pltpu.SemaphoreType.DMA,
            pltpu.SemaphoreType.DMA,
            pltpu.SemaphoreType.DMA,
        ],
    )
    return f(nodes_p, nidx_p, emb)[:2]


def _tc_body(w1_ref, w2_ref, xs_ref, xn_ref, o_ref):
    a = lax.dot_general(w1_ref[...], xs_ref[...],
                        (((1,), (1,)), ((), ())),
                        preferred_element_type=jnp.float32)
    b = lax.dot_general(w2_ref[...], xn_ref[...],
                        (((1,), (1,)), ((), ())),
                        preferred_element_type=jnp.float32)
    o_ref[...] = jnp.maximum(a + b, 0.0)


def _tc_combine(w1, w2, xs, xn, n_out):
    blk = 2048
    grid = _BP // blk
    return pl.pallas_call(
        _tc_body,
        grid=(grid,),
        in_specs=[
            pl.BlockSpec((_D, _D), lambda i: (0, 0)),
            pl.BlockSpec((_D, _D), lambda i: (0, 0)),
            pl.BlockSpec((blk, _D), lambda i: (i, 0)),
            pl.BlockSpec((blk, _D), lambda i: (i, 0)),
        ],
        out_specs=pl.BlockSpec((_D, blk), lambda i: (0, i)),
        out_shape=jax.ShapeDtypeStruct((_D, n_out), jnp.float32),
    )(w1, w2, xs, xn)


def kernel(nodes, emb, neigh_idx, W):
    B = nodes.shape[0]
    nodes32 = nodes.astype(jnp.int32)
    nidx32 = neigh_idx.astype(jnp.int32)
    nidx_p = jnp.pad(nidx32, ((0, 0), (0, 128 - _K)))
    nodes_p = jnp.zeros((_BP,), jnp.int32).at[:B].set(nodes32)
    self_f, neigh_s = _sc_gather(nodes_p, nidx_p, emb)
    w1 = W[:, :_D]
    w2 = W[:, _D:] * (1.0 / _K)
    return _tc_combine(w1, w2, self_f, neigh_s, B)


# speedup vs baseline: 1.3242x; 1.0102x over previous
import functools

import numpy as np

import jax
import jax.numpy as jnp
from jax import lax
from jax.experimental import pallas as pl
from jax.experimental.pallas import tpu as pltpu
from jax.experimental.pallas import tpu_sc as plsc

_D = 256
_K = 16
_NC = 2
_NS = 16
_NW = _NC * _NS
_BP = 10240
_BW = _BP // _NW
_CH = 8
_NCHUNK = _BW // _CH
_NG = _D // 16


def _sc_body(nodes_hbm, nidx_hbm, emb_hbm, self_hbm, neigh_hbm, embb_hbm,
             nodes_v, nids2_v, nidsf_v, rows0_v, rows1_v, obuf0_v, obuf1_v,
             selfbuf_v, selfbuf2_v, pbin_v, pbout_v,
             sem_n, sem_s, sem_g0, sem_g1, sem_o0, sem_o1):
    c = lax.axis_index("c")
    s = lax.axis_index("s")
    wid = c * _NS + s
    base = wid * _BW

    n_chunks = emb_hbm.shape[0] // 40
    with jax.named_scope("sc_pack"):
        def _pk(j, _):
            cid = j * _NS + s

            @pl.when(cid < n_chunks)
            def _():
                r0 = cid * 40
                pltpu.sync_copy(emb_hbm.at[pl.ds(r0, 40)], pbin_v)

                def _prow(i, _):
                    for cc in range(_D // 32):
                        a = pbin_v[i, pl.ds(cc * 32, 16)]
                        bq = pbin_v[i, pl.ds(cc * 32 + 16, 16)]
                        pk = plsc.pack(a, bq,
                                       format=plsc.PackFormat.INTERLEAVED)
                        pbout_v[i, pl.ds(cc * 16, 16)] = plsc.bitcast(
                            pk, jnp.int32)
                    return 0
                lax.fori_loop(0, 40, _prow, 0)
                pltpu.sync_copy(pbout_v, embb_hbm.at[c, pl.ds(r0, 40)])
            return 0
        lax.fori_loop(0, (n_chunks + _NS - 1) // _NS, _pk, 0)

    with jax.named_scope("sc_nids"):
        pltpu.sync_copy(nodes_hbm.at[pl.ds(base, _BW)], nodes_v)
        for k in range(4):
            pltpu.async_copy(
                nidx_hbm.at[nodes_v.at[pl.ds(k * 80, 80)]], nids2_v, sem_n
            ).wait()

            def _flat(i, _):
                nidsf_v[pl.ds((k * 80 + i) * _K, _K)] = (
                    nids2_v[i, pl.ds(0, _K)])
                return 0
            lax.fori_loop(0, 80, _flat, 0)

    with jax.named_scope("sc_self"):
        sbufs = (selfbuf_v, selfbuf2_v)
        pltpu.async_copy(
            emb_hbm.at[nodes_v.at[pl.ds(0, 80)]], sbufs[0], sem_s)
        for k in range(4):
            pltpu.make_async_copy(
                emb_hbm.at[nodes_v.at[pl.ds(k * 80, 80)]], sbufs[k % 2],
                sem_s).wait()
            if k < 3:
                pltpu.async_copy(
                    emb_hbm.at[nodes_v.at[pl.ds((k + 1) * 80, 80)]],
                    sbufs[(k + 1) % 2], sem_s)
            pltpu.sync_copy(sbufs[k % 2],
                            self_hbm.at[pl.ds(base + k * 80, 80)])

    rows = (rows0_v, rows1_v)
    obufs = (obuf0_v, obuf1_v)
    sems = (sem_g0, sem_g1)
    sems_o = (sem_o0, sem_o1)

    def _fire(ci, b):
        return pltpu.async_copy(
            embb_hbm.at[c].at[nidsf_v.at[pl.ds(ci * (_CH * _K), _CH * _K)]],
            rows[b], sems[b])

    with jax.named_scope("sc_prime"):
        plsc.subcore_barrier()
        _fire(0, 0)
        _fire(1, 1)

    def _pair(p, _):
        for b in range(2):
            ci = p * 2 + b
            @pl.when(ci >= 1)
            def _():
                pltpu.async_copy(
                    obufs[1 - b],
                    neigh_hbm.at[pl.ds(base + (ci - 1) * _CH, _CH)],
                    sems_o[1 - b])
            pltpu.make_async_copy(
                embb_hbm.at[c].at[
                    nidsf_v.at[pl.ds(ci * (_CH * _K), _CH * _K)]],
                rows[b], sems[b]).wait()
            @pl.when(ci >= 2)
            def _():
                pltpu.make_async_copy(
                    obufs[b], neigh_hbm.at[pl.ds(base, _CH)], sems_o[b]
                ).wait()

            def _node(j, _):
                r0 = j * _K
                for cc in range(_D // 32):

                    def _tree(v):
                        while len(v) > 1:
                            v = [v[2 * i] + v[2 * i + 1]
                                 for i in range(len(v) // 2)]
                        return v[0]
                    evens, odds = [], []
                    for r in range(_K):
                        w = rows[b][r0 + r, pl.ds(cc * 16, 16)]
                        e, o = plsc.unpack(
                            plsc.bitcast(w, jnp.bfloat16),
                            format=plsc.PackFormat.INTERLEAVED)
                        evens.append(e)
                        odds.append(o)
                    obufs[b][j, pl.ds(cc * 32, 16)] = _tree(evens)
                    obufs[b][j, pl.ds(cc * 32 + 16, 16)] = _tree(odds)
                return 0
            lax.fori_loop(0, _CH, _node, 0)

            @pl.when(ci + 2 < _NCHUNK)
            def _():
                _fire(ci + 2, b)
        return 0
    with jax.named_scope("sc_main"):
        lax.fori_loop(0, _NCHUNK // 2, _pair, 0)

    with jax.named_scope("sc_drain"):
        pltpu.make_async_copy(
            obufs[0], neigh_hbm.at[pl.ds(base, _CH)], sems_o[0]).wait()
        pl.delay(256)
        pltpu.sync_copy(
            obufs[1],
            neigh_hbm.at[pl.ds(base + (_NCHUNK - 1) * _CH, _CH)])


def _sc_gather(nodes_p, nidx_p, emb):
    mesh = plsc.VectorSubcoreMesh(core_axis_name="c", subcore_axis_name="s")
    f = pl.kernel(
        _sc_body,
        out_type=(
            jax.ShapeDtypeStruct((_BP, _D), jnp.float32),
            jax.ShapeDtypeStruct((_BP, _D), jnp.float32),
            jax.ShapeDtypeStruct((_NC, emb.shape[0], _D // 2), jnp.int32),
        ),
        mesh=mesh,
        compiler_params=pltpu.CompilerParams(needs_layout_passes=False),
        scratch_types=[
            pltpu.VMEM((_BW,), jnp.int32),
            pltpu.VMEM((80, 128), jnp.int32),
            pltpu.VMEM((_BW * _K,), jnp.int32),
            pltpu.VMEM((_CH * _K, _D // 2), jnp.int32),
            pltpu.VMEM((_CH * _K, _D // 2), jnp.int32),
            pltpu.VMEM((_CH, _D), jnp.float32),
            pltpu.VMEM((_CH, _D), jnp.float32),
            pltpu.VMEM((80, _D), jnp.float32),
            pltpu.VMEM((80, _D), jnp.float32),
            pltpu.VMEM((40, _D), jnp.float32),
            pltpu.VMEM((40, _D // 2), jnp.int32),
            pltpu.SemaphoreType.DMA,
            pltpu.SemaphoreType.DMA,
            pltpu.SemaphoreType.DMA,
            pltpu.SemaphoreType.DMA,
            pltpu.SemaphoreType.DMA,
            pltpu.SemaphoreType.DMA,
        ],
    )
    return f(nodes_p, nidx_p, emb)[:2]


def _tc_body(w1_ref, w2_ref, xs_ref, xn_ref, o_ref):
    a = lax.dot_general(w1_ref[...], xs_ref[...],
                        (((1,), (1,)), ((), ())),
                        preferred_element_type=jnp.float32)
    b = lax.dot_general(w2_ref[...], xn_ref[...],
                        (((1,), (1,)), ((), ())),
                        preferred_element_type=jnp.float32)
    o_ref[...] = jnp.maximum(a + b, 0.0)


def _tc_combine(w1, w2, xs, xn, n_out):
    blk = 2048
    grid = _BP // blk
    return pl.pallas_call(
        _tc_body,
        grid=(grid,),
        in_specs=[
            pl.BlockSpec((_D, _D), lambda i: (0, 0)),
            pl.BlockSpec((_D, _D), lambda i: (0, 0)),
            pl.BlockSpec((blk, _D), lambda i: (i, 0)),
            pl.BlockSpec((blk, _D), lambda i: (i, 0)),
        ],
        out_specs=pl.BlockSpec((_D, blk), lambda i: (0, i)),
        out_shape=jax.ShapeDtypeStruct((_D, n_out), jnp.float32),
    )(w1, w2, xs, xn)


def kernel(nodes, emb, neigh_idx, W):
    B = nodes.shape[0]
    nodes32 = nodes.astype(jnp.int32)
    nidx32 = neigh_idx.astype(jnp.int32)
    nidx_p = jnp.pad(nidx32, ((0, 0), (0, 128 - _K)))
    nodes_p = jnp.zeros((_BP,), jnp.int32).at[:B].set(nodes32)
    self_f, neigh_s = _sc_gather(nodes_p, nidx_p, emb)
    w1 = W[:, :_D]
    w2 = W[:, _D:] * (1.0 / _K)
    return _tc_combine(w1, w2, self_f, neigh_s, B)
